# baseline (speedup 1.0000x reference)
import jax
import jax.numpy as jnp
import numpy as np
from jax import lax
from jax.experimental import pallas as pl
from jax.experimental.pallas import tpu as pltpu
from jax.experimental.pallas import tpu_sc as plsc

_L = 20
_B = 16384
_NC = 2
_NS = 16
_NW = _NC * _NS
_BPW = _B // _NW
_GRP = _BPW // 16
_CHUNK = 128
_NCHUNK = _BPW // _CHUNK

_DELTA = np.float32(1e-15)
_PI = np.float32(3.14159265358979)
_SQRTHF = np.float32(0.70710678118654752440)
_LN2_HI = np.float32(0.693359375)
_LN2_LO = np.float32(-2.12194440e-4)
_LOG_COEFFS = (
    np.float32(-1.1514610310e-1), np.float32(1.1676998740e-1),
    np.float32(-1.2420140846e-1), np.float32(1.4249322787e-1),
    np.float32(-1.6668057665e-1), np.float32(2.0000714765e-1),
    np.float32(-2.4999993993e-1), np.float32(3.3333331174e-1),
)


def _log_angle(v):
    x = jnp.abs(v + _DELTA)
    bits = lax.bitcast_convert_type(x, jnp.int32)
    e = lax.shift_right_logical(bits, 23) - 126
    m = lax.bitcast_convert_type(
        (bits & np.int32(0x007FFFFF)) | np.int32(0x3F000000), jnp.float32)
    small = m < _SQRTHF
    e = jnp.where(small, e - 1, e)
    m = jnp.where(small, m + m, m)
    t = m - np.float32(1.0)
    z = t * t
    p = np.float32(7.0376836292e-2)
    for c in _LOG_COEFFS:
        p = p * t + c
    ef = e.astype(jnp.float32)
    y = t * z * p
    y = y + ef * _LN2_LO
    y = y - np.float32(0.5) * z
    re = t + y + ef * _LN2_HI
    im = jnp.where(v < np.float32(0.0), _PI, np.float32(0.0))
    return re, im


def _body(s_hbm, w_hbm, re_hbm, im_hbm, s_v, idx_v, vals_v, re_v, im_v, sem):
    wid = lax.axis_index("s") * _NC + lax.axis_index("c")
    base = wid * _BPW
    pltpu.sync_copy(s_hbm.at[:, pl.ds(base, _BPW)], s_v)

    def idx_body(g, carry):
        off = g * 16
        acc = s_v[0, pl.ds(off, 16)]
        for l in range(1, _L):
            acc = acc + acc + s_v[l, pl.ds(off, 16)]
        idx_v[pl.ds(off, 16)] = acc
        return carry

    lax.fori_loop(0, _GRP, idx_body, 0)

    copies = [
        pltpu.async_copy(w_hbm.at[idx_v.at[pl.ds(j * _CHUNK, _CHUNK)]],
                         vals_v.at[pl.ds(j * _CHUNK, _CHUNK)], sem)
        for j in range(_NCHUNK)
    ]
    for cp in copies:
        cp.wait()

    def math_body(g, carry):
        off = g * 16
        re, im = _log_angle(vals_v[pl.ds(off, 16)])
        re_v[pl.ds(off, 16)] = re
        im_v[pl.ds(off, 16)] = im
        return carry

    lax.fori_loop(0, _GRP, math_body, 0)

    pltpu.sync_copy(re_v, re_hbm.at[pl.ds(base, _BPW)])
    pltpu.sync_copy(im_v, im_hbm.at[pl.ds(base, _BPW)])


def kernel(s, w):
    mesh = plsc.VectorSubcoreMesh(core_axis_name="c", subcore_axis_name="s")
    re, im = pl.kernel(
        _body,
        out_type=[
            jax.ShapeDtypeStruct((_B,), jnp.float32),
            jax.ShapeDtypeStruct((_B,), jnp.float32),
        ],
        mesh=mesh,
        scratch_types=[
            pltpu.VMEM((_L, _BPW), jnp.int32),
            pltpu.VMEM((_BPW,), jnp.int32),
            pltpu.VMEM((_BPW,), jnp.float32),
            pltpu.VMEM((_BPW,), jnp.float32),
            pltpu.VMEM((_BPW,), jnp.float32),
            pltpu.SemaphoreType.DMA,
        ],
    )(s, w)
    return lax.complex(re, im)
